# Initial kernel scaffold; baseline (speedup 1.0000x reference)
import jax
import jax.numpy as jnp
from jax.experimental import pallas as pl


def kernel(feature):
    raise NotImplementedError("write your pallas kernel here")


# speedup vs baseline: 2.2987x; 2.2987x over previous
import functools
import random

import jax
import jax.numpy as jnp
import numpy as np
from jax import lax
from jax.experimental import pallas as pl
from jax.experimental.pallas import tpu as pltpu
from jax.experimental.pallas import tpu_sc as plsc

_LANES = 16
_NUM_CORES = 2
_NUM_SUBCORES = 16
_NUM_WORKERS = _NUM_CORES * _NUM_SUBCORES


def _distortion_indices(h, w, a_max=3, r_max=0.7):
    random.seed(0)
    cols = h
    rows = w
    center_rows = int(np.round(random.uniform(1, rows - 2)))
    center_cols = int(np.round(random.uniform(1, cols - 2)))
    radius = random.uniform(0.03 * max(rows, cols), r_max * max(rows, cols))
    choice = random.randint(0, 1)
    spect_ratio1 = 1
    spect_ratio2 = 1
    if choice == 1:
        spect_ratio1 = random.uniform(1, a_max)
    else:
        spect_ratio2 = random.uniform(1, a_max)
    cols_np = np.arange(cols)
    rows_np = np.arange(rows)
    cols_np_t = np.tile(cols_np, (rows, 1))
    cols_pow = np.power(cols_np_t - center_cols, 2)
    rows_np_t = np.tile(rows_np, (cols, 1))
    rows_pow = np.power(rows_np_t - center_rows, 2)
    dis = np.sqrt(cols_pow + rows_pow.transpose())
    judge = (spect_ratio1 * np.abs(rows_np_t - center_rows).transpose()
             + spect_ratio2 * np.abs(cols_np_t - center_cols))
    index = np.where(judge <= radius)
    index_rows = np.rint(index[0]).astype('int64')
    index_cols = np.rint(index[1]).astype('int64')
    dis_val = dis[index]
    old_i = np.floor(dis_val / radius * (index_rows - center_rows)
                     + center_rows).astype('int64')
    old_j = np.floor(dis_val / radius * (index_cols - center_cols)
                     + center_cols).astype('int64')
    return index_rows, index_cols, old_i, old_j


@functools.lru_cache(maxsize=None)
def _index_plan(h, w):
    ir, ic, oi, oj = _distortion_indices(h, w)
    src = (oi * w + oj).astype(np.int32)
    dst = (ir * w + ic).astype(np.int32)
    k = len(src)
    kpad = ((k + _LANES - 1) // _LANES) * _LANES
    assert 0 not in set(dst.tolist())
    pad_src = np.zeros(kpad - k, dtype=np.int32)
    pad_dst = np.zeros(kpad - k, dtype=np.int32)
    return np.concatenate([src, pad_src]), np.concatenate([dst, pad_dst])


def _sc_permute(planes, src_off, dst_off):
    num_planes, hw = planes.shape
    kpad = src_off.shape[0]
    ppw = num_planes // _NUM_WORKERS
    assert num_planes % _NUM_WORKERS == 0
    nvec = kpad // _LANES

    mesh = plsc.VectorSubcoreMesh(
        core_axis_name="c", subcore_axis_name="s",
        num_cores=_NUM_CORES, num_subcores=_NUM_SUBCORES)

    @functools.partial(
        pl.kernel,
        out_type=jax.ShapeDtypeStruct((num_planes, hw), jnp.float32),
        mesh=mesh,
        scratch_types=[
            pltpu.VMEM((hw,), jnp.float32),
            pltpu.VMEM((kpad,), jnp.int32),
            pltpu.VMEM((kpad,), jnp.int32),
            pltpu.VMEM((kpad,), jnp.float32),
        ],
        compiler_params=pltpu.CompilerParams(needs_layout_passes=False),
    )
    def body(feat_hbm, src_hbm, dst_hbm, out_hbm, buf, srcv, dstv, vals):
        wid = lax.axis_index("s") * _NUM_CORES + lax.axis_index("c")
        pltpu.sync_copy(src_hbm, srcv)
        pltpu.sync_copy(dst_hbm, dstv)

        def plane_body(i, carry):
            p = wid * ppw + i
            pltpu.sync_copy(feat_hbm.at[p], buf)

            def gat(j, c):
                idx = srcv[pl.ds(j * _LANES, _LANES)]
                vals[pl.ds(j * _LANES, _LANES)] = plsc.load_gather(buf, [idx])
                return c

            lax.fori_loop(0, nvec, gat, 0)

            def sca(j, c):
                idx = dstv[pl.ds(j * _LANES, _LANES)]
                plsc.store_scatter(buf, [idx],
                                   vals[pl.ds(j * _LANES, _LANES)])
                return c

            lax.fori_loop(0, nvec, sca, 0)
            pltpu.sync_copy(buf, out_hbm.at[p])
            return carry

        lax.fori_loop(0, ppw, plane_body, 0)

    return body(planes, src_off, dst_off)


def kernel(feature):
    b, c, h, w = feature.shape
    src_np, dst_np = _index_plan(h, w)
    src_off = jnp.asarray(src_np)
    dst_off = jnp.asarray(dst_np)
    planes = feature.reshape(b * c, h * w)
    out = _sc_permute(planes, src_off, dst_off)
    return out.reshape(b, c, h, w)
